# Initial kernel scaffold; baseline (speedup 1.0000x reference)
import jax
import jax.numpy as jnp
from jax.experimental import pallas as pl


def kernel(inputs):
    raise NotImplementedError("write your pallas kernel here")


# speedup vs baseline: 5.2353x; 5.2353x over previous
import jax
import jax.numpy as jnp
from jax.experimental import pallas as pl
from jax.experimental.pallas import tpu as pltpu

_W = 256
_S = 16
_NCHUNK = _W // _S


def _roll_rows(x, s):
    return jnp.concatenate([x[s:], x[:s]], axis=0)


def _bitonic_sort_rows(x, iota_rows):
    n = x.shape[0]
    k = 2
    while k <= n:
        j = k // 2
        while j >= 1:
            low = (iota_rows & j) == 0
            take_min = low == ((iota_rows & k) == 0)
            partner = jnp.where(low, _roll_rows(x, j), _roll_rows(x, n - j))
            x = jnp.where(take_min, jnp.minimum(x, partner),
                          jnp.maximum(x, partner))
            j //= 2
        k *= 2
    return x


def _stats_body(d_ref, o_ref):
    nwpad = o_ref.shape[3]
    d = d_ref[0, 0]
    win = jnp.concatenate([d[:, a:a + nwpad] for a in range(_NCHUNK)], axis=0)

    inv_w = 1.0 / _W
    mean = jnp.sum(win, axis=0, keepdims=True) * inv_w
    cen = win - mean
    var = jnp.sum(cen * cen, axis=0, keepdims=True) * inv_w
    std = jnp.sqrt(var)
    mn = jnp.min(win, axis=0, keepdims=True)
    mx = jnp.max(win, axis=0, keepdims=True)
    one = jnp.float32(1.0)
    zero = jnp.float32(0.0)
    above = jnp.sum(jnp.where(win > mean, one, zero), axis=0, keepdims=True)
    below = jnp.sum(jnp.where(win < mean, one, zero), axis=0, keepdims=True)

    iota_rows = jax.lax.broadcasted_iota(jnp.int32, win.shape, 0)
    s = _bitonic_sort_rows(win, iota_rows)
    q25 = 0.25 * s[63:64] + 0.75 * s[64:65]
    med = 0.5 * (s[127:128] + s[128:129])
    q75 = 0.75 * s[191:192] + 0.25 * s[192:193]
    iqr = q75 - q25

    o_ref[0, 0] = jnp.concatenate(
        [mean, std, mn, mx, med, iqr, above, below], axis=0)


def kernel(inputs):
    B, T, F = inputs.shape
    nw = (T - _W) // _S + 1
    nwpad = ((nw + 127) // 128) * 128
    ch = nwpad + _NCHUNK

    xt = jnp.transpose(inputs, (0, 2, 1))
    xp = jnp.pad(xt, ((0, 0), (0, 0), (0, ch * _S - T)))
    d = jnp.transpose(xp.reshape(B, F, ch, _S), (0, 1, 3, 2))

    out = pl.pallas_call(
        _stats_body,
        grid=(B, F),
        in_specs=[pl.BlockSpec((1, 1, _S, ch), lambda b, f: (b, f, 0, 0))],
        out_specs=pl.BlockSpec((1, 1, 8, nwpad), lambda b, f: (b, f, 0, 0)),
        out_shape=jax.ShapeDtypeStruct((B, F, 8, nwpad), jnp.float32),
        compiler_params=pltpu.CompilerParams(
            dimension_semantics=("parallel", "parallel")),
    )(d)

    return jnp.transpose(out, (0, 3, 1, 2)).reshape(B, nwpad, F * 8)[:, :nw]


# speedup vs baseline: 10.2161x; 1.9514x over previous
import jax
import jax.numpy as jnp
from jax.experimental import pallas as pl
from jax.experimental.pallas import tpu as pltpu

_W = 256
_S = 16
_NCHUNK = _W // _S
_TILE = 128


def _roll_rows(x, s):
    return jnp.concatenate([x[s:], x[:s]], axis=0)


def _cx_stage(x, k, j, iota, desc=False):
    n = x.shape[0]
    if j >= 8:
        nb = n // j
        partner = jnp.concatenate(
            [x[(b ^ 1) * j:((b ^ 1) + 1) * j] for b in range(nb)], axis=0)
        mn = jnp.minimum(x, partner)
        mx = jnp.maximum(x, partner)
        pieces = []
        for b in range(nb):
            up = ((b * j) & k) == 0
            if desc:
                up = not up
            take_min = (b % 2 == 0) == up
            src = mn if take_min else mx
            pieces.append(src[b * j:(b + 1) * j])
        return jnp.concatenate(pieces, axis=0)
    low = (iota & j) == 0
    up = (iota & k) == 0
    take_min = (low != up) if desc else (low == up)
    partner = jnp.where(low, _roll_rows(x, j), _roll_rows(x, n - j))
    return jnp.where(take_min, jnp.minimum(x, partner),
                     jnp.maximum(x, partner))


def _sort16_both(x, iota16):
    for k in (2, 4, 8):
        j = k // 2
        while j >= 1:
            x = _cx_stage(x, k, j, iota16)
            j //= 2
    asc = dsc = x
    for j in (8, 4, 2, 1):
        asc = _cx_stage(asc, 16, j, iota16)
        dsc = _cx_stage(dsc, 16, j, iota16, desc=True)
    return asc, dsc


def _stats_body(d_ref, o_ref):
    nwpad = o_ref.shape[3]
    d = d_ref[0, 0]
    iota16 = jax.lax.broadcasted_iota(jnp.int32, d.shape, 0)
    dasc, ddsc = _sort16_both(d, iota16)
    srcs = [dasc if a % 2 == 0 else ddsc for a in range(_NCHUNK)]

    inv_w = 1.0 / _W
    one = jnp.float32(1.0)
    zero = jnp.float32(0.0)
    iota256 = jax.lax.broadcasted_iota(jnp.int32, (_W, _TILE), 0)

    for t in range(nwpad // _TILE):
        c0 = t * _TILE
        win = jnp.concatenate(
            [srcs[a][:, a + c0:a + c0 + _TILE] for a in range(_NCHUNK)],
            axis=0)

        mean = jnp.sum(win, axis=0, keepdims=True) * inv_w
        cen = win - mean
        var = jnp.sum(cen * cen, axis=0, keepdims=True) * inv_w
        std = jnp.sqrt(var)
        mn = jnp.min(win, axis=0, keepdims=True)
        mx = jnp.max(win, axis=0, keepdims=True)
        above = jnp.sum(jnp.where(win > mean, one, zero), axis=0,
                        keepdims=True)
        below = jnp.sum(jnp.where(win < mean, one, zero), axis=0,
                        keepdims=True)

        s = win
        for k in (32, 64, 128, 256):
            j = k // 2
            while j >= 1:
                s = _cx_stage(s, k, j, iota256)
                j //= 2
        q25 = 0.25 * s[63:64] + 0.75 * s[64:65]
        med = 0.5 * (s[127:128] + s[128:129])
        q75 = 0.75 * s[191:192] + 0.25 * s[192:193]
        iqr = q75 - q25

        o_ref[0, 0, :, c0:c0 + _TILE] = jnp.concatenate(
            [mean, std, mn, mx, med, iqr, above, below], axis=0)


def kernel(inputs):
    B, T, F = inputs.shape
    nw = (T - _W) // _S + 1
    nwpad = ((nw + _TILE - 1) // _TILE) * _TILE
    ch = nwpad + _NCHUNK

    xt = jnp.transpose(inputs, (0, 2, 1))
    xp = jnp.pad(xt, ((0, 0), (0, 0), (0, ch * _S - T)))
    d = jnp.transpose(xp.reshape(B, F, ch, _S), (0, 1, 3, 2))

    out = pl.pallas_call(
        _stats_body,
        grid=(B, F),
        in_specs=[pl.BlockSpec((1, 1, _S, ch), lambda b, f: (b, f, 0, 0))],
        out_specs=pl.BlockSpec((1, 1, 8, nwpad), lambda b, f: (b, f, 0, 0)),
        out_shape=jax.ShapeDtypeStruct((B, F, 8, nwpad), jnp.float32),
        compiler_params=pltpu.CompilerParams(
            dimension_semantics=("parallel", "parallel")),
    )(d)

    return jnp.transpose(out, (0, 3, 1, 2)).reshape(B, nwpad, F * 8)[:, :nw]


# speedup vs baseline: 13.9410x; 1.3646x over previous
import jax
import jax.numpy as jnp
from jax.experimental import pallas as pl
from jax.experimental.pallas import tpu as pltpu

_W = 256
_S = 16
_NCHUNK = _W // _S
_TILE = 128


def _roll_rows(x, s):
    return jnp.concatenate([x[s:], x[:s]], axis=0)


def _lshift(x, s):
    return jnp.concatenate([x[:, s:], x[:, :s]], axis=1)


def _cx_asc(x, j, row_off=0):
    n = x.shape[0]
    if j >= 8:
        nb = n // j
        b0 = row_off // j
        partner = jnp.concatenate(
            [x[(b ^ 1) * j:((b ^ 1) + 1) * j] for b in range(nb)], axis=0)
        mn = jnp.minimum(x, partner)
        mx = jnp.maximum(x, partner)
        pieces = [
            (mn if (b + b0) % 2 == 0 else mx)[b * j:(b + 1) * j]
            for b in range(nb)
        ]
        return jnp.concatenate(pieces, axis=0)
    iota = jax.lax.broadcasted_iota(jnp.int32, x.shape, 0)
    low = (iota & j) == 0
    return jnp.where(low, jnp.minimum(x, _roll_rows(x, j)),
                     jnp.maximum(x, _roll_rows(x, n - j)))


def _cx_directed(x, k, j, iota):
    low = (iota & j) == 0
    take_min = low == ((iota & k) == 0)
    partner = jnp.where(low, _roll_rows(x, j), _roll_rows(x, x.shape[0] - j))
    return jnp.where(take_min, jnp.minimum(x, partner),
                     jnp.maximum(x, partner))


def _sort16_stacked(v):
    iota = jax.lax.broadcasted_iota(jnp.int32, v.shape, 0)
    for k in (2, 4, 8):
        j = k // 2
        while j >= 1:
            v = _cx_directed(v, k, j, iota)
            j //= 2
    for j in (8, 4, 2, 1):
        v = _cx_asc(v, j)
    return v


def _stats_body(d_ref, o_ref):
    nwpad = o_ref.shape[3]
    d = d_ref[0, 0]

    v = _sort16_stacked(jnp.concatenate([d, -d], axis=0))
    p1, n1 = v[:16], v[16:]

    v = jnp.concatenate(
        [p1, -_lshift(n1, 1), n1, -_lshift(p1, 1)], axis=0)
    for j in (16, 8, 4, 2, 1):
        v = _cx_asc(v, j)
    p32, n32 = v[:32], v[32:]

    v = jnp.concatenate(
        [p32, -_lshift(n32, 2), n32, -_lshift(p32, 2)], axis=0)
    for j in (32, 16, 8, 4, 2, 1):
        v = _cx_asc(v, j)
    p64, n64 = v[:64], v[64:]

    v = jnp.concatenate(
        [p64, -_lshift(n64, 4), n64, -_lshift(p64, 4)], axis=0)
    for j in (64, 32, 16, 8, 4, 2, 1):
        v = _cx_asc(v, j)
    p128, n128 = v[:128], v[128:]

    hi128 = -_lshift(n128, 8)

    inv_w = 1.0 / _W
    one = jnp.float32(1.0)
    zero = jnp.float32(0.0)

    for t in range(nwpad // _TILE):
        c0 = t * _TILE
        lo = p128[:, c0:c0 + _TILE]
        hi = hi128[:, c0:c0 + _TILE]
        s = jnp.concatenate([lo, hi], axis=0)

        mean = jnp.sum(s, axis=0, keepdims=True) * inv_w
        cen = s - mean
        var = jnp.sum(cen * cen, axis=0, keepdims=True) * inv_w
        std = jnp.sqrt(var)
        mn = jnp.minimum(s[0:1], s[255:256])
        mx = jnp.maximum(s[127:128], s[128:129])
        above = jnp.sum(jnp.where(s > mean, one, zero), axis=0, keepdims=True)
        below = jnp.sum(jnp.where(s < mean, one, zero), axis=0, keepdims=True)

        for j in (128, 64, 32):
            s = _cx_asc(s, j)
        s = s[32:224]
        s = _cx_asc(s, 16, row_off=32)
        s = s[16:176]
        for j in (8, 4, 2, 1):
            s = _cx_asc(s, j, row_off=48)
        q25 = 0.25 * s[15:16] + 0.75 * s[16:17]
        med = 0.5 * (s[79:80] + s[80:81])
        q75 = 0.75 * s[143:144] + 0.25 * s[144:145]
        iqr = q75 - q25

        o_ref[0, 0, :, c0:c0 + _TILE] = jnp.concatenate(
            [mean, std, mn, mx, med, iqr, above, below], axis=0)


def kernel(inputs):
    B, T, F = inputs.shape
    nw = (T - _W) // _S + 1
    nwpad = ((nw + _TILE - 1) // _TILE) * _TILE
    ch = nwpad + _NCHUNK

    xt = jnp.transpose(inputs, (0, 2, 1))
    xp = jnp.pad(xt, ((0, 0), (0, 0), (0, ch * _S - T)))
    d = jnp.transpose(xp.reshape(B, F, ch, _S), (0, 1, 3, 2))

    out = pl.pallas_call(
        _stats_body,
        grid=(B, F),
        in_specs=[pl.BlockSpec((1, 1, _S, ch), lambda b, f: (b, f, 0, 0))],
        out_specs=pl.BlockSpec((1, 1, 8, nwpad), lambda b, f: (b, f, 0, 0)),
        out_shape=jax.ShapeDtypeStruct((B, F, 8, nwpad), jnp.float32),
        compiler_params=pltpu.CompilerParams(
            dimension_semantics=("parallel", "parallel")),
    )(d)

    return jnp.transpose(out, (0, 3, 1, 2)).reshape(B, nwpad, F * 8)[:, :nw]


# speedup vs baseline: 16.6722x; 1.1959x over previous
import jax
import jax.numpy as jnp
from jax.experimental import pallas as pl
from jax.experimental.pallas import tpu as pltpu

_W = 256
_S = 16
_NCHUNK = _W // _S
_TILE = 128


def _roll_rows(x, s):
    return jnp.concatenate([x[s:], x[:s]], axis=0)


def _lshift(x, s):
    return jnp.concatenate([x[:, s:], x[:, :s]], axis=1)


def _cx_asc(x, j, row_off=0):
    n = x.shape[0]
    if j >= 8:
        nb = n // j
        b0 = row_off // j
        partner = jnp.concatenate(
            [x[(b ^ 1) * j:((b ^ 1) + 1) * j] for b in range(nb)], axis=0)
        mn = jnp.minimum(x, partner)
        mx = jnp.maximum(x, partner)
        pieces = [
            (mn if (b + b0) % 2 == 0 else mx)[b * j:(b + 1) * j]
            for b in range(nb)
        ]
        return jnp.concatenate(pieces, axis=0)
    iota = jax.lax.broadcasted_iota(jnp.int32, x.shape, 0)
    low = (iota & j) == 0
    return jnp.where(low, jnp.minimum(x, _roll_rows(x, j)),
                     jnp.maximum(x, _roll_rows(x, n - j)))


def _cx_directed(x, k, j, iota):
    low = (iota & j) == 0
    take_min = low == ((iota & k) == 0)
    partner = jnp.where(low, _roll_rows(x, j), _roll_rows(x, x.shape[0] - j))
    return jnp.where(take_min, jnp.minimum(x, partner),
                     jnp.maximum(x, partner))


def _sort16_stacked(v):
    iota = jax.lax.broadcasted_iota(jnp.int32, v.shape, 0)
    for k in (2, 4, 8):
        j = k // 2
        while j >= 1:
            v = _cx_directed(v, k, j, iota)
            j //= 2
    for j in (8, 4, 2, 1):
        v = _cx_asc(v, j)
    return v


def _stats_body(d_ref, o_ref):
    nwpad = o_ref.shape[3]
    d = d_ref[0, 0]

    v = _sort16_stacked(jnp.concatenate([d, -d], axis=0))
    p1, n1 = v[:16], v[16:]

    v = jnp.concatenate(
        [p1, -_lshift(n1, 1), n1, -_lshift(p1, 1)], axis=0)
    for j in (16, 8, 4, 2, 1):
        v = _cx_asc(v, j)
    p32, n32 = v[:32], v[32:]

    v = jnp.concatenate(
        [p32, -_lshift(n32, 2), n32, -_lshift(p32, 2)], axis=0)
    for j in (32, 16, 8, 4, 2, 1):
        v = _cx_asc(v, j)
    p64, n64 = v[:64], v[64:]

    v = jnp.concatenate(
        [p64, -_lshift(n64, 4), n64, -_lshift(p64, 4)], axis=0)
    for j in (64, 32, 16, 8, 4, 2, 1):
        v = _cx_asc(v, j)
    p128, n128 = v[:128], v[128:]

    hi128 = -_lshift(n128, 8)

    inv_w = 1.0 / _W
    one = jnp.float32(1.0)
    zero = jnp.float32(0.0)

    for t in range(nwpad // _TILE):
        c0 = t * _TILE
        lo = p128[:, c0:c0 + _TILE]
        hi = hi128[:, c0:c0 + _TILE]
        s = jnp.concatenate([lo, hi], axis=0)

        mean = jnp.sum(s, axis=0, keepdims=True) * inv_w
        cen = s - mean
        var = jnp.sum(cen * cen, axis=0, keepdims=True) * inv_w
        std = jnp.sqrt(var)
        mn = jnp.minimum(s[0:1], s[255:256])
        mx = jnp.maximum(s[127:128], s[128:129])
        above = jnp.sum(jnp.where(s > mean, one, zero), axis=0, keepdims=True)
        below = jnp.sum(jnp.where(s < mean, one, zero), axis=0, keepdims=True)

        for j in (128, 64, 32):
            s = _cx_asc(s, j)
        s = s[32:224]
        s = _cx_asc(s, 16, row_off=32)
        s = s[16:176]
        s = _cx_asc(s, 8, row_off=48)
        s = jnp.concatenate([s[8:24], s[72:88], s[136:152]], axis=0)
        for j in (4, 2, 1):
            s = _cx_asc(s, j, row_off=56)
        q25 = 0.25 * s[7:8] + 0.75 * s[8:9]
        med = 0.5 * (s[23:24] + s[24:25])
        q75 = 0.75 * s[39:40] + 0.25 * s[40:41]
        iqr = q75 - q25

        o_ref[0, 0, :, c0:c0 + _TILE] = jnp.concatenate(
            [mean, std, mn, mx, med, iqr, above, below], axis=0)


def kernel(inputs):
    B, T, F = inputs.shape
    nw = (T - _W) // _S + 1
    nwpad = ((nw + _TILE - 1) // _TILE) * _TILE
    ch = max(nwpad, -(-T // _S))

    xt = jnp.transpose(inputs, (0, 2, 1))
    xp = jnp.pad(xt, ((0, 0), (0, 0), (0, ch * _S - T)))
    d = jnp.transpose(xp.reshape(B, F, ch, _S), (0, 1, 3, 2))

    out = pl.pallas_call(
        _stats_body,
        grid=(B, F),
        in_specs=[pl.BlockSpec((1, 1, _S, ch), lambda b, f: (b, f, 0, 0))],
        out_specs=pl.BlockSpec((1, 1, 8, nwpad), lambda b, f: (b, f, 0, 0)),
        out_shape=jax.ShapeDtypeStruct((B, F, 8, nwpad), jnp.float32),
        compiler_params=pltpu.CompilerParams(
            dimension_semantics=("parallel", "parallel")),
    )(d)

    return jnp.transpose(out, (0, 3, 1, 2)).reshape(B, nwpad, F * 8)[:, :nw]


# speedup vs baseline: 17.8039x; 1.0679x over previous
import jax
import jax.numpy as jnp
from jax.experimental import pallas as pl
from jax.experimental.pallas import tpu as pltpu

_W = 256
_S = 16
_NCHUNK = _W // _S
_TILE = 128


def _roll_rows(x, s):
    return jnp.concatenate([x[s:], x[:s]], axis=0)


def _lshift(x, s):
    return jnp.concatenate([x[:, s:], x[:, :s]], axis=1)


def _cx_asc(x, j, row_off=0):
    n = x.shape[0]
    if j >= 8:
        nb = n // j
        b0 = row_off // j
        partner = jnp.concatenate(
            [x[(b ^ 1) * j:((b ^ 1) + 1) * j] for b in range(nb)], axis=0)
        mn = jnp.minimum(x, partner)
        mx = jnp.maximum(x, partner)
        pieces = [
            (mn if (b + b0) % 2 == 0 else mx)[b * j:(b + 1) * j]
            for b in range(nb)
        ]
        return jnp.concatenate(pieces, axis=0)
    iota = jax.lax.broadcasted_iota(jnp.int32, x.shape, 0)
    low = (iota & j) == 0
    return jnp.where(low, jnp.minimum(x, _roll_rows(x, j)),
                     jnp.maximum(x, _roll_rows(x, n - j)))


def _cx_directed(x, k, j, iota):
    low = (iota & j) == 0
    take_min = low == ((iota & k) == 0)
    partner = jnp.where(low, _roll_rows(x, j), _roll_rows(x, x.shape[0] - j))
    return jnp.where(take_min, jnp.minimum(x, partner),
                     jnp.maximum(x, partner))


def _sort16_stacked(v):
    iota = jax.lax.broadcasted_iota(jnp.int32, v.shape, 0)
    for k in (2, 4, 8):
        j = k // 2
        while j >= 1:
            v = _cx_directed(v, k, j, iota)
            j //= 2
    for j in (8, 4, 2, 1):
        v = _cx_asc(v, j)
    return v


def _stats_body(d_ref, o_ref):
    for f in range(d_ref.shape[1]):
        _stats_one_channel(d_ref, o_ref, f)


def _stats_one_channel(d_ref, o_ref, f):
    nwpad = o_ref.shape[3]
    d = d_ref[0, f]

    v = _sort16_stacked(jnp.concatenate([d, -d], axis=0))
    p1, n1 = v[:16], v[16:]

    v = jnp.concatenate(
        [p1, -_lshift(n1, 1), n1, -_lshift(p1, 1)], axis=0)
    for j in (16, 8, 4, 2, 1):
        v = _cx_asc(v, j)
    p32, n32 = v[:32], v[32:]

    v = jnp.concatenate(
        [p32, -_lshift(n32, 2), n32, -_lshift(p32, 2)], axis=0)
    for j in (32, 16, 8, 4, 2, 1):
        v = _cx_asc(v, j)
    p64, n64 = v[:64], v[64:]

    v = jnp.concatenate(
        [p64, -_lshift(n64, 4), n64, -_lshift(p64, 4)], axis=0)
    for j in (64, 32, 16, 8, 4, 2, 1):
        v = _cx_asc(v, j)
    p128, n128 = v[:128], v[128:]

    hi128 = -_lshift(n128, 8)

    inv_w = 1.0 / _W
    one = jnp.float32(1.0)
    zero = jnp.float32(0.0)

    for t in range(nwpad // _TILE):
        c0 = t * _TILE
        lo = p128[:, c0:c0 + _TILE]
        hi = hi128[:, c0:c0 + _TILE]
        s = jnp.concatenate([lo, hi], axis=0)

        mean = jnp.sum(s, axis=0, keepdims=True) * inv_w
        cen = s - mean
        var = jnp.sum(cen * cen, axis=0, keepdims=True) * inv_w
        std = jnp.sqrt(var)
        mn = jnp.minimum(s[0:1], s[255:256])
        mx = jnp.maximum(s[127:128], s[128:129])
        above = jnp.sum(jnp.where(s > mean, one, zero), axis=0, keepdims=True)
        below = jnp.sum(jnp.where(s < mean, one, zero), axis=0, keepdims=True)

        for j in (128, 64, 32):
            s = _cx_asc(s, j)
        s = s[32:224]
        s = _cx_asc(s, 16, row_off=32)
        s = s[16:176]
        s = _cx_asc(s, 8, row_off=48)
        s = jnp.concatenate([s[8:24], s[72:88], s[136:152]], axis=0)
        for j in (4, 2, 1):
            s = _cx_asc(s, j, row_off=56)
        q25 = 0.25 * s[7:8] + 0.75 * s[8:9]
        med = 0.5 * (s[23:24] + s[24:25])
        q75 = 0.75 * s[39:40] + 0.25 * s[40:41]
        iqr = q75 - q25

        o_ref[0, f, :, c0:c0 + _TILE] = jnp.concatenate(
            [mean, std, mn, mx, med, iqr, above, below], axis=0)


def kernel(inputs):
    B, T, F = inputs.shape
    nw = (T - _W) // _S + 1
    nwpad = ((nw + _TILE - 1) // _TILE) * _TILE
    ch = max(nwpad, -(-T // _S))

    xt = jnp.transpose(inputs, (0, 2, 1))
    xp = jnp.pad(xt, ((0, 0), (0, 0), (0, ch * _S - T)))
    d = jnp.transpose(xp.reshape(B, F, ch, _S), (0, 1, 3, 2))

    out = pl.pallas_call(
        _stats_body,
        grid=(B,),
        in_specs=[pl.BlockSpec((1, F, _S, ch), lambda b: (b, 0, 0, 0))],
        out_specs=pl.BlockSpec((1, F, 8, nwpad), lambda b: (b, 0, 0, 0)),
        out_shape=jax.ShapeDtypeStruct((B, F, 8, nwpad), jnp.float32),
        compiler_params=pltpu.CompilerParams(
            dimension_semantics=("parallel",)),
    )(d)

    return jnp.transpose(out, (0, 3, 1, 2)).reshape(B, nwpad, F * 8)[:, :nw]


# speedup vs baseline: 21.4943x; 1.2073x over previous
import jax
import jax.numpy as jnp
from jax.experimental import pallas as pl
from jax.experimental.pallas import tpu as pltpu

_W = 256
_S = 16
_NCHUNK = _W // _S
_TILE = 128


def _roll_rows(x, s):
    return jnp.concatenate([x[s:], x[:s]], axis=0)


def _lshift(x, s):
    return jnp.concatenate([x[:, s:], x[:, :s]], axis=1)


def _cx_asc(x, j, row_off=0):
    n = x.shape[0]
    if j >= 8:
        nb = n // j
        b0 = row_off // j
        partner = jnp.concatenate(
            [x[(b ^ 1) * j:((b ^ 1) + 1) * j] for b in range(nb)], axis=0)
        mn = jnp.minimum(x, partner)
        mx = jnp.maximum(x, partner)
        pieces = [
            (mn if (b + b0) % 2 == 0 else mx)[b * j:(b + 1) * j]
            for b in range(nb)
        ]
        return jnp.concatenate(pieces, axis=0)
    iota = jax.lax.broadcasted_iota(jnp.int32, x.shape, 0)
    low = (iota & j) == 0
    return jnp.where(low, jnp.minimum(x, _roll_rows(x, j)),
                     jnp.maximum(x, _roll_rows(x, n - j)))


def _cx_directed(x, k, j, iota):
    low = (iota & j) == 0
    take_min = low == ((iota & k) == 0)
    partner = jnp.where(low, _roll_rows(x, j), _roll_rows(x, x.shape[0] - j))
    return jnp.where(take_min, jnp.minimum(x, partner),
                     jnp.maximum(x, partner))


def _sort16_stacked(v):
    iota = jax.lax.broadcasted_iota(jnp.int32, v.shape, 0)
    for k in (2, 4, 8):
        j = k // 2
        while j >= 1:
            v = _cx_directed(v, k, j, iota)
            j //= 2
    for j in (8, 4, 2, 1):
        v = _cx_asc(v, j)
    return v


def _stats_body(d_ref, o_ref):
    for f in range(d_ref.shape[1]):
        _stats_one_channel(d_ref, o_ref, f)


def _rev_rows(x):
    n = x.shape[0]
    x = jnp.concatenate(
        [x[m * 8:(m + 1) * 8] for m in reversed(range(n // 8))], axis=0)
    iota = jax.lax.broadcasted_iota(jnp.int32, x.shape, 0)
    for dd in (4, 2, 1):
        x = jnp.where((iota & dd) == 0, _roll_rows(x, dd),
                      _roll_rows(x, n - dd))
    return x


def _stats_one_channel(d_ref, o_ref, f):
    nwpad = o_ref.shape[3]
    d = d_ref[0, f]

    p1 = _sort16_stacked(d)

    v = jnp.concatenate([p1, _rev_rows(_lshift(p1, 1))], axis=0)
    for j in (16, 8, 4, 2, 1):
        v = _cx_asc(v, j)
    p32 = v

    v = jnp.concatenate([p32, _rev_rows(_lshift(p32, 2))], axis=0)
    for j in (32, 16, 8, 4, 2, 1):
        v = _cx_asc(v, j)
    p64 = v

    v = jnp.concatenate([p64, _rev_rows(_lshift(p64, 4))], axis=0)
    for j in (64, 32, 16, 8, 4, 2, 1):
        v = _cx_asc(v, j)
    p128 = v

    hi128 = _rev_rows(_lshift(p128, 8))

    inv_w = 1.0 / _W
    one = jnp.float32(1.0)
    zero = jnp.float32(0.0)

    for t in range(nwpad // _TILE):
        c0 = t * _TILE
        lo = p128[:, c0:c0 + _TILE]
        hi = hi128[:, c0:c0 + _TILE]
        s = jnp.concatenate([lo, hi], axis=0)

        mean = jnp.sum(s, axis=0, keepdims=True) * inv_w
        cen = s - mean
        var = jnp.sum(cen * cen, axis=0, keepdims=True) * inv_w
        std = jnp.sqrt(var)
        mn = jnp.minimum(s[0:1], s[255:256])
        mx = jnp.maximum(s[127:128], s[128:129])
        above = jnp.sum(jnp.where(s > mean, one, zero), axis=0, keepdims=True)
        below = jnp.sum(jnp.where(s < mean, one, zero), axis=0, keepdims=True)

        for j in (128, 64, 32):
            s = _cx_asc(s, j)
        s = s[32:224]
        s = _cx_asc(s, 16, row_off=32)
        s = s[16:176]
        s = _cx_asc(s, 8, row_off=48)
        s = jnp.concatenate([s[8:24], s[72:88], s[136:152]], axis=0)
        for j in (4, 2, 1):
            s = _cx_asc(s, j, row_off=56)
        q25 = 0.25 * s[7:8] + 0.75 * s[8:9]
        med = 0.5 * (s[23:24] + s[24:25])
        q75 = 0.75 * s[39:40] + 0.25 * s[40:41]
        iqr = q75 - q25

        o_ref[0, f, :, c0:c0 + _TILE] = jnp.concatenate(
            [mean, std, mn, mx, med, iqr, above, below], axis=0)


def kernel(inputs):
    B, T, F = inputs.shape
    nw = (T - _W) // _S + 1
    nwpad = ((nw + _TILE - 1) // _TILE) * _TILE
    ch = max(nwpad, -(-T // _S))

    xt = jnp.transpose(inputs, (0, 2, 1))
    xp = jnp.pad(xt, ((0, 0), (0, 0), (0, ch * _S - T)))
    d = jnp.transpose(xp.reshape(B, F, ch, _S), (0, 1, 3, 2))

    out = pl.pallas_call(
        _stats_body,
        grid=(B,),
        in_specs=[pl.BlockSpec((1, F, _S, ch), lambda b: (b, 0, 0, 0))],
        out_specs=pl.BlockSpec((1, F, 8, nwpad), lambda b: (b, 0, 0, 0)),
        out_shape=jax.ShapeDtypeStruct((B, F, 8, nwpad), jnp.float32),
        compiler_params=pltpu.CompilerParams(
            dimension_semantics=("parallel",)),
    )(d)

    return jnp.transpose(out, (0, 3, 1, 2)).reshape(B, nwpad, F * 8)[:, :nw]


# speedup vs baseline: 21.8653x; 1.0173x over previous
import jax
import jax.numpy as jnp
from jax.experimental import pallas as pl
from jax.experimental.pallas import tpu as pltpu

_W = 256
_S = 16
_NCHUNK = _W // _S
_TILE = 128


def _roll_rows(x, s):
    return jnp.concatenate([x[s:], x[:s]], axis=0)


def _lshift(x, s):
    return pltpu.roll(x, x.shape[1] - s, axis=1)


def _cx_asc(x, j, row_off=0):
    n = x.shape[0]
    if j >= 8:
        nb = n // j
        b0 = row_off // j
        partner = jnp.concatenate(
            [x[(b ^ 1) * j:((b ^ 1) + 1) * j] for b in range(nb)], axis=0)
        mn = jnp.minimum(x, partner)
        mx = jnp.maximum(x, partner)
        pieces = [
            (mn if (b + b0) % 2 == 0 else mx)[b * j:(b + 1) * j]
            for b in range(nb)
        ]
        return jnp.concatenate(pieces, axis=0)
    iota = jax.lax.broadcasted_iota(jnp.int32, x.shape, 0)
    low = (iota & j) == 0
    return jnp.where(low, jnp.minimum(x, _roll_rows(x, j)),
                     jnp.maximum(x, _roll_rows(x, n - j)))


def _cx_directed(x, k, j, iota):
    low = (iota & j) == 0
    take_min = low == ((iota & k) == 0)
    partner = jnp.where(low, _roll_rows(x, j), _roll_rows(x, x.shape[0] - j))
    return jnp.where(take_min, jnp.minimum(x, partner),
                     jnp.maximum(x, partner))


def _sort16_stacked(v):
    iota = jax.lax.broadcasted_iota(jnp.int32, v.shape, 0)
    for k in (2, 4, 8):
        j = k // 2
        while j >= 1:
            v = _cx_directed(v, k, j, iota)
            j //= 2
    for j in (8, 4, 2, 1):
        v = _cx_asc(v, j)
    return v


def _stats_body(d_ref, o_ref):
    for f in range(d_ref.shape[1]):
        _stats_one_channel(d_ref, o_ref, f)


def _rev_rows(x):
    n = x.shape[0]
    x = jnp.concatenate(
        [x[m * 8:(m + 1) * 8] for m in reversed(range(n // 8))], axis=0)
    iota = jax.lax.broadcasted_iota(jnp.int32, x.shape, 0)
    for dd in (4, 2, 1):
        x = jnp.where((iota & dd) == 0, _roll_rows(x, dd),
                      _roll_rows(x, n - dd))
    return x


def _stats_one_channel(d_ref, o_ref, f):
    nwpad = o_ref.shape[3]
    d = d_ref[0, f]

    p1 = _sort16_stacked(d)

    v = jnp.concatenate([p1, _rev_rows(_lshift(p1, 1))], axis=0)
    for j in (16, 8, 4, 2, 1):
        v = _cx_asc(v, j)
    p32 = v

    v = jnp.concatenate([p32, _rev_rows(_lshift(p32, 2))], axis=0)
    for j in (32, 16, 8, 4, 2, 1):
        v = _cx_asc(v, j)
    p64 = v

    v = jnp.concatenate([p64, _rev_rows(_lshift(p64, 4))], axis=0)
    for j in (64, 32, 16, 8, 4, 2, 1):
        v = _cx_asc(v, j)
    p128 = v

    hi128 = _rev_rows(_lshift(p128, 8))

    inv_w = 1.0 / _W
    one = jnp.float32(1.0)
    zero = jnp.float32(0.0)

    for t in range(nwpad // _TILE):
        c0 = t * _TILE
        lo = p128[:, c0:c0 + _TILE]
        hi = hi128[:, c0:c0 + _TILE]
        s = jnp.concatenate([lo, hi], axis=0)

        mean = jnp.sum(s, axis=0, keepdims=True) * inv_w
        cen = s - mean
        var = jnp.sum(cen * cen, axis=0, keepdims=True) * inv_w
        std = jnp.sqrt(var)
        mn = jnp.minimum(s[0:1], s[255:256])
        mx = jnp.maximum(s[127:128], s[128:129])
        above = jnp.sum(jnp.where(s > mean, one, zero), axis=0, keepdims=True)
        below = jnp.sum(jnp.where(s < mean, one, zero), axis=0, keepdims=True)

        for j in (128, 64, 32):
            s = _cx_asc(s, j)
        s = s[32:224]
        s = _cx_asc(s, 16, row_off=32)
        s = s[16:176]
        s = _cx_asc(s, 8, row_off=48)
        s = jnp.concatenate([s[8:24], s[72:88], s[136:152]], axis=0)
        for j in (4, 2, 1):
            s = _cx_asc(s, j, row_off=56)
        q25 = 0.25 * s[7:8] + 0.75 * s[8:9]
        med = 0.5 * (s[23:24] + s[24:25])
        q75 = 0.75 * s[39:40] + 0.25 * s[40:41]
        iqr = q75 - q25

        o_ref[0, f, :, c0:c0 + _TILE] = jnp.concatenate(
            [mean, std, mn, mx, med, iqr, above, below], axis=0)


def kernel(inputs):
    B, T, F = inputs.shape
    nw = (T - _W) // _S + 1
    nwpad = ((nw + _TILE - 1) // _TILE) * _TILE
    ch = max(nwpad, -(-T // _S))

    xt = jnp.transpose(inputs, (0, 2, 1))
    xp = jnp.pad(xt, ((0, 0), (0, 0), (0, ch * _S - T)))
    d = jnp.transpose(xp.reshape(B, F, ch, _S), (0, 1, 3, 2))

    out = pl.pallas_call(
        _stats_body,
        grid=(B,),
        in_specs=[pl.BlockSpec((1, F, _S, ch), lambda b: (b, 0, 0, 0))],
        out_specs=pl.BlockSpec((1, F, 8, nwpad), lambda b: (b, 0, 0, 0)),
        out_shape=jax.ShapeDtypeStruct((B, F, 8, nwpad), jnp.float32),
        compiler_params=pltpu.CompilerParams(
            dimension_semantics=("parallel",)),
    )(d)

    return jnp.transpose(out, (0, 3, 1, 2)).reshape(B, nwpad, F * 8)[:, :nw]


# speedup vs baseline: 21.8784x; 1.0006x over previous
import jax
import jax.numpy as jnp
from jax.experimental import pallas as pl
from jax.experimental.pallas import tpu as pltpu

_W = 256
_S = 16
_NCHUNK = _W // _S
_TILE = 128


def _roll_rows(x, s):
    return jnp.concatenate([x[s:], x[:s]], axis=0)


def _lshift(x, s):
    return pltpu.roll(x, x.shape[1] - s, axis=1)


def _cx_asc(x, j, row_off=0):
    n = x.shape[0]
    if j >= 8:
        nb = n // j
        b0 = row_off // j
        partner = jnp.concatenate(
            [x[(b ^ 1) * j:((b ^ 1) + 1) * j] for b in range(nb)], axis=0)
        mn = jnp.minimum(x, partner)
        mx = jnp.maximum(x, partner)
        pieces = [
            (mn if (b + b0) % 2 == 0 else mx)[b * j:(b + 1) * j]
            for b in range(nb)
        ]
        return jnp.concatenate(pieces, axis=0)
    iota = jax.lax.broadcasted_iota(jnp.int32, x.shape, 0)
    low = (iota & j) == 0
    return jnp.where(low, jnp.minimum(x, _roll_rows(x, j)),
                     jnp.maximum(x, _roll_rows(x, n - j)))


def _cx_directed(x, k, j, iota):
    low = (iota & j) == 0
    take_min = low == ((iota & k) == 0)
    partner = jnp.where(low, _roll_rows(x, j), _roll_rows(x, x.shape[0] - j))
    return jnp.where(take_min, jnp.minimum(x, partner),
                     jnp.maximum(x, partner))


def _sort16_stacked(v):
    iota = jax.lax.broadcasted_iota(jnp.int32, v.shape, 0)
    for k in (2, 4, 8):
        j = k // 2
        while j >= 1:
            v = _cx_directed(v, k, j, iota)
            j //= 2
    for j in (8, 4, 2, 1):
        v = _cx_asc(v, j)
    return v


def _stats_body(d_ref, o_ref):
    for f in range(d_ref.shape[1]):
        _stats_one_channel(d_ref, o_ref, f)


def _rev_rows(x):
    n = x.shape[0]
    x = jnp.concatenate(
        [x[m * 8:(m + 1) * 8] for m in reversed(range(n // 8))], axis=0)
    iota = jax.lax.broadcasted_iota(jnp.int32, x.shape, 0)
    for dd in (4, 2, 1):
        x = jnp.where((iota & dd) == 0, _roll_rows(x, dd),
                      _roll_rows(x, n - dd))
    return x


def _stats_one_channel(d_ref, o_ref, f):
    nwpad = o_ref.shape[3]
    d = d_ref[0, f]

    p1 = _sort16_stacked(d)

    v = jnp.concatenate([p1, _rev_rows(_lshift(p1, 1))], axis=0)
    for j in (16, 8, 4, 2, 1):
        v = _cx_asc(v, j)
    p32 = v

    v = jnp.concatenate([p32, _rev_rows(_lshift(p32, 2))], axis=0)
    for j in (32, 16, 8, 4, 2, 1):
        v = _cx_asc(v, j)
    p64 = v

    v = jnp.concatenate([p64, _rev_rows(_lshift(p64, 4))], axis=0)
    for j in (64, 32, 16, 8, 4, 2, 1):
        v = _cx_asc(v, j)
    p128 = v

    hi128 = _rev_rows(_lshift(p128, 8))

    inv_w = 1.0 / _W
    one = jnp.float32(1.0)
    zero = jnp.float32(0.0)

    for t in range(nwpad // _TILE):
        c0 = t * _TILE
        lo = p128[:, c0:c0 + _TILE]
        hi = hi128[:, c0:c0 + _TILE]
        s = jnp.concatenate([lo, hi], axis=0)

        mean = jnp.sum(s, axis=0, keepdims=True) * inv_w
        cen = s - mean
        var = jnp.sum(cen * cen, axis=0, keepdims=True) * inv_w
        std = jnp.sqrt(var)
        mn = jnp.minimum(s[0:1], s[255:256])
        mx = jnp.maximum(s[127:128], s[128:129])
        above = jnp.sum(jnp.where(s > mean, one, zero), axis=0, keepdims=True)
        below = jnp.sum(jnp.where(s < mean, one, zero), axis=0, keepdims=True)

        for j in (128, 64, 32):
            s = _cx_asc(s, j)
        s = s[32:224]
        s = _cx_asc(s, 16, row_off=32)
        s = s[16:176]
        s = _cx_asc(s, 8, row_off=48)
        s = jnp.concatenate([s[8:24], s[72:88], s[136:152]], axis=0)
        for j in (4, 2, 1):
            s = _cx_asc(s, j, row_off=56)
        q25 = 0.25 * s[7:8] + 0.75 * s[8:9]
        med = 0.5 * (s[23:24] + s[24:25])
        q75 = 0.75 * s[39:40] + 0.25 * s[40:41]
        iqr = q75 - q25

        o_ref[0, f, :, c0:c0 + _TILE] = jnp.concatenate(
            [mean, std, mn, mx, med, iqr, above, below], axis=0)


def kernel(inputs):
    B, T, F = inputs.shape
    nw = (T - _W) // _S + 1
    nwpad = ((nw + _TILE - 1) // _TILE) * _TILE
    ch = max(nwpad, -(-T // _S))

    xp = jnp.pad(inputs, ((0, 0), (0, ch * _S - T), (0, 0)))
    d = jnp.transpose(xp.reshape(B, ch, _S, F), (0, 3, 2, 1))

    out = pl.pallas_call(
        _stats_body,
        grid=(B,),
        in_specs=[pl.BlockSpec((1, F, _S, ch), lambda b: (b, 0, 0, 0))],
        out_specs=pl.BlockSpec((1, F, 8, nwpad), lambda b: (b, 0, 0, 0)),
        out_shape=jax.ShapeDtypeStruct((B, F, 8, nwpad), jnp.float32),
        compiler_params=pltpu.CompilerParams(
            dimension_semantics=("parallel",)),
    )(d)

    return jnp.transpose(out, (0, 3, 1, 2)).reshape(B, nwpad, F * 8)[:, :nw]


# speedup vs baseline: 21.9348x; 1.0026x over previous
import jax
import jax.numpy as jnp
from jax.experimental import pallas as pl
from jax.experimental.pallas import tpu as pltpu

_W = 256
_S = 16
_NCHUNK = _W // _S
_TILE = 128


def _roll_rows(x, s):
    return pltpu.roll(x, x.shape[0] - s, axis=0)


def _lshift(x, s):
    return pltpu.roll(x, x.shape[1] - s, axis=1)


def _cx_asc(x, j, row_off=0):
    n = x.shape[0]
    if j >= 8:
        nb = n // j
        b0 = row_off // j
        partner = jnp.concatenate(
            [x[(b ^ 1) * j:((b ^ 1) + 1) * j] for b in range(nb)], axis=0)
        mn = jnp.minimum(x, partner)
        mx = jnp.maximum(x, partner)
        pieces = [
            (mn if (b + b0) % 2 == 0 else mx)[b * j:(b + 1) * j]
            for b in range(nb)
        ]
        return jnp.concatenate(pieces, axis=0)
    iota = jax.lax.broadcasted_iota(jnp.int32, x.shape, 0)
    low = (iota & j) == 0
    return jnp.where(low, jnp.minimum(x, _roll_rows(x, j)),
                     jnp.maximum(x, _roll_rows(x, n - j)))


def _cx_directed(x, k, j, iota):
    low = (iota & j) == 0
    take_min = low == ((iota & k) == 0)
    partner = jnp.where(low, _roll_rows(x, j), _roll_rows(x, x.shape[0] - j))
    return jnp.where(take_min, jnp.minimum(x, partner),
                     jnp.maximum(x, partner))


def _sort16_stacked(v):
    iota = jax.lax.broadcasted_iota(jnp.int32, v.shape, 0)
    for k in (2, 4, 8):
        j = k // 2
        while j >= 1:
            v = _cx_directed(v, k, j, iota)
            j //= 2
    for j in (8, 4, 2, 1):
        v = _cx_asc(v, j)
    return v


def _stats_body(d_ref, o_ref):
    for f in range(d_ref.shape[1]):
        _stats_one_channel(d_ref, o_ref, f)


def _rev_rows(x):
    n = x.shape[0]
    x = jnp.concatenate(
        [x[m * 8:(m + 1) * 8] for m in reversed(range(n // 8))], axis=0)
    iota = jax.lax.broadcasted_iota(jnp.int32, x.shape, 0)
    for dd in (4, 2, 1):
        x = jnp.where((iota & dd) == 0, _roll_rows(x, dd),
                      _roll_rows(x, n - dd))
    return x


def _stats_one_channel(d_ref, o_ref, f):
    nwpad = o_ref.shape[3]
    d = d_ref[0, f]

    p1 = _sort16_stacked(d)

    v = jnp.concatenate([p1, _rev_rows(_lshift(p1, 1))], axis=0)
    for j in (16, 8, 4, 2, 1):
        v = _cx_asc(v, j)
    p32 = v

    v = jnp.concatenate([p32, _rev_rows(_lshift(p32, 2))], axis=0)
    for j in (32, 16, 8, 4, 2, 1):
        v = _cx_asc(v, j)
    p64 = v

    v = jnp.concatenate([p64, _rev_rows(_lshift(p64, 4))], axis=0)
    for j in (64, 32, 16, 8, 4, 2, 1):
        v = _cx_asc(v, j)
    p128 = v

    hi128 = _rev_rows(_lshift(p128, 8))

    inv_w = 1.0 / _W
    one = jnp.float32(1.0)
    zero = jnp.float32(0.0)

    for t in range(nwpad // _TILE):
        c0 = t * _TILE
        lo = p128[:, c0:c0 + _TILE]
        hi = hi128[:, c0:c0 + _TILE]
        s = jnp.concatenate([lo, hi], axis=0)

        mean = jnp.sum(s, axis=0, keepdims=True) * inv_w
        cen = s - mean
        var = jnp.sum(cen * cen, axis=0, keepdims=True) * inv_w
        std = jnp.sqrt(var)
        mn = jnp.minimum(s[0:1], s[255:256])
        mx = jnp.maximum(s[127:128], s[128:129])
        above = jnp.sum(jnp.where(s > mean, one, zero), axis=0, keepdims=True)
        below = jnp.sum(jnp.where(s < mean, one, zero), axis=0, keepdims=True)

        for j in (128, 64, 32):
            s = _cx_asc(s, j)
        s = s[32:224]
        s = _cx_asc(s, 16, row_off=32)
        s = s[16:176]
        s = _cx_asc(s, 8, row_off=48)
        s = jnp.concatenate([s[8:24], s[72:88], s[136:152]], axis=0)
        for j in (4, 2, 1):
            s = _cx_asc(s, j, row_off=56)
        q25 = 0.25 * s[7:8] + 0.75 * s[8:9]
        med = 0.5 * (s[23:24] + s[24:25])
        q75 = 0.75 * s[39:40] + 0.25 * s[40:41]
        iqr = q75 - q25

        o_ref[0, f, :, c0:c0 + _TILE] = jnp.concatenate(
            [mean, std, mn, mx, med, iqr, above, below], axis=0)


def kernel(inputs):
    B, T, F = inputs.shape
    nw = (T - _W) // _S + 1
    nwpad = ((nw + _TILE - 1) // _TILE) * _TILE
    ch = max(nwpad, -(-T // _S))

    xp = jnp.pad(inputs, ((0, 0), (0, ch * _S - T), (0, 0)))
    d = jnp.transpose(xp.reshape(B, ch, _S, F), (0, 3, 2, 1))

    out = pl.pallas_call(
        _stats_body,
        grid=(B,),
        in_specs=[pl.BlockSpec((1, F, _S, ch), lambda b: (b, 0, 0, 0))],
        out_specs=pl.BlockSpec((1, F, 8, nwpad), lambda b: (b, 0, 0, 0)),
        out_shape=jax.ShapeDtypeStruct((B, F, 8, nwpad), jnp.float32),
        compiler_params=pltpu.CompilerParams(
            dimension_semantics=("parallel",),
            allow_input_fusion=[True]),
    )(d)

    return jnp.transpose(out, (0, 3, 1, 2)).reshape(B, nwpad, F * 8)[:, :nw]


# speedup vs baseline: 22.5572x; 1.0284x over previous
import jax
import jax.numpy as jnp
from jax.experimental import pallas as pl
from jax.experimental.pallas import tpu as pltpu

_W = 256
_S = 16
_NCHUNK = _W // _S
_TILE = 128


def _roll_rows(x, s):
    return pltpu.roll(x, x.shape[0] - s, axis=0)


def _lshift(x, s):
    return pltpu.roll(x, x.shape[1] - s, axis=1)


def _cx_asc(x, j, row_off=0):
    n = x.shape[0]
    if j >= 8:
        nb = n // j
        b0 = row_off // j
        partner = jnp.concatenate(
            [x[(b ^ 1) * j:((b ^ 1) + 1) * j] for b in range(nb)], axis=0)
        mn = jnp.minimum(x, partner)
        mx = jnp.maximum(x, partner)
        pieces = [
            (mn if (b + b0) % 2 == 0 else mx)[b * j:(b + 1) * j]
            for b in range(nb)
        ]
        return jnp.concatenate(pieces, axis=0)
    iota = jax.lax.broadcasted_iota(jnp.int32, x.shape, 0)
    low = (iota & j) == 0
    return jnp.where(low, jnp.minimum(x, _roll_rows(x, j)),
                     jnp.maximum(x, _roll_rows(x, n - j)))


def _cx_directed(x, k, j, iota):
    low = (iota & j) == 0
    take_min = low == ((iota & k) == 0)
    partner = jnp.where(low, _roll_rows(x, j), _roll_rows(x, x.shape[0] - j))
    return jnp.where(take_min, jnp.minimum(x, partner),
                     jnp.maximum(x, partner))


def _sort16_stacked(v):
    iota = jax.lax.broadcasted_iota(jnp.int32, v.shape, 0)
    for k in (2, 4, 8):
        j = k // 2
        while j >= 1:
            v = _cx_directed(v, k, j, iota)
            j //= 2
    for j in (8, 4, 2, 1):
        v = _cx_asc(v, j)
    return v


def _stats_body(d_ref, o_ref):
    for f in range(d_ref.shape[1]):
        _stats_one_channel(d_ref, o_ref, f)


def _rev_rows(x):
    n = x.shape[0]
    x = jnp.concatenate(
        [x[m * 8:(m + 1) * 8] for m in reversed(range(n // 8))], axis=0)
    iota = jax.lax.broadcasted_iota(jnp.int32, x.shape, 0)
    for dd in (4, 2, 1):
        x = jnp.where((iota & dd) == 0, _roll_rows(x, dd),
                      _roll_rows(x, n - dd))
    return x


def _stats_one_channel(d_ref, o_ref, f):
    nwpad = o_ref.shape[3]
    d = d_ref[0, f]

    p1 = _sort16_stacked(d)

    v = jnp.concatenate([p1, _rev_rows(_lshift(p1, 1))], axis=0)
    for j in (16, 8, 4, 2, 1):
        v = _cx_asc(v, j)
    p32 = v

    v = jnp.concatenate([p32, _rev_rows(_lshift(p32, 2))], axis=0)
    for j in (32, 16, 8, 4, 2, 1):
        v = _cx_asc(v, j)
    p64 = v

    v = jnp.concatenate([p64, _rev_rows(_lshift(p64, 4))], axis=0)
    for j in (64, 32, 16, 8, 4, 2, 1):
        v = _cx_asc(v, j)
    p128 = v

    hi128 = _rev_rows(_lshift(p128, 8))

    inv_w = 1.0 / _W
    one = jnp.float32(1.0)
    zero = jnp.float32(0.0)

    s1 = jnp.sum(d, axis=0, keepdims=True)
    q1 = jnp.sum(d * d, axis=0, keepdims=True)
    for sh in (1, 2, 4, 8):
        s1 = s1 + _lshift(s1, sh)
        q1 = q1 + _lshift(q1, sh)
    mean_full = s1 * inv_w
    var_full = jnp.maximum(q1 * inv_w - mean_full * mean_full, zero)
    std_full = jnp.sqrt(var_full)

    for t in range(nwpad // _TILE):
        c0 = t * _TILE
        lo = p128[:, c0:c0 + _TILE]
        hi = hi128[:, c0:c0 + _TILE]
        s = jnp.concatenate([lo, hi], axis=0)

        mean = mean_full[:, c0:c0 + _TILE]
        std = std_full[:, c0:c0 + _TILE]
        mn = jnp.minimum(s[0:1], s[255:256])
        mx = jnp.maximum(s[127:128], s[128:129])
        above = jnp.sum(jnp.where(s > mean, one, zero), axis=0, keepdims=True)
        below = jnp.sum(jnp.where(s < mean, one, zero), axis=0, keepdims=True)

        for j in (128, 64, 32):
            s = _cx_asc(s, j)
        s = s[32:224]
        s = _cx_asc(s, 16, row_off=32)
        s = s[16:176]
        s = _cx_asc(s, 8, row_off=48)
        s = jnp.concatenate([s[8:24], s[72:88], s[136:152]], axis=0)
        for j in (4, 2, 1):
            s = _cx_asc(s, j, row_off=56)
        q25 = 0.25 * s[7:8] + 0.75 * s[8:9]
        med = 0.5 * (s[23:24] + s[24:25])
        q75 = 0.75 * s[39:40] + 0.25 * s[40:41]
        iqr = q75 - q25

        o_ref[0, f, :, c0:c0 + _TILE] = jnp.concatenate(
            [mean, std, mn, mx, med, iqr, above, below], axis=0)


def kernel(inputs):
    B, T, F = inputs.shape
    nw = (T - _W) // _S + 1
    nwpad = ((nw + _TILE - 1) // _TILE) * _TILE
    ch = max(nwpad, -(-T // _S))

    xp = jnp.pad(inputs, ((0, 0), (0, ch * _S - T), (0, 0)))
    d = jnp.transpose(xp.reshape(B, ch, _S, F), (0, 3, 2, 1))

    out = pl.pallas_call(
        _stats_body,
        grid=(B,),
        in_specs=[pl.BlockSpec((1, F, _S, ch), lambda b: (b, 0, 0, 0))],
        out_specs=pl.BlockSpec((1, F, 8, nwpad), lambda b: (b, 0, 0, 0)),
        out_shape=jax.ShapeDtypeStruct((B, F, 8, nwpad), jnp.float32),
        compiler_params=pltpu.CompilerParams(
            dimension_semantics=("parallel",),
            allow_input_fusion=[True]),
    )(d)

    return jnp.transpose(out, (0, 3, 1, 2)).reshape(B, nwpad, F * 8)[:, :nw]
